# Initial kernel scaffold; baseline (speedup 1.0000x reference)
import jax
import jax.numpy as jnp
from jax.experimental import pallas as pl


def kernel(input_ids, table, soft_prompt):
    raise NotImplementedError("write your pallas kernel here")


# speedup vs baseline: 1.1788x; 1.1788x over previous
import functools

import jax
import jax.numpy as jnp
from jax import lax
from jax.experimental import pallas as pl
from jax.experimental.pallas import tpu as pltpu
from jax.experimental.pallas import tpu_sc as plsc

_VOCAB = 100000
_D = 128
_NP = 20
_B = 4
_T = 2048

_NC = 2
_NS = 16
_NW = _NC * _NS
_BT = _B * _T
_PER_W = _BT // _NW
_IDX_CHUNK = 128
_N_CHUNKS = _PER_W // _IDX_CHUNK
_OUT_ROWS = _B * (_NP + _T)

_mesh = plsc.VectorSubcoreMesh(
    core_axis_name="c", subcore_axis_name="s", num_cores=_NC, num_subcores=_NS
)


@functools.partial(
    pl.kernel,
    out_type=jax.ShapeDtypeStruct((_OUT_ROWS, _D), jnp.float32),
    mesh=_mesh,
    scratch_types=[
        pltpu.VMEM((_N_CHUNKS, _IDX_CHUNK), jnp.int32),
        pltpu.VMEM((_PER_W, _D), jnp.float32),
        pltpu.VMEM((_NP, _D), jnp.float32),
        pltpu.SemaphoreType.DMA,
        pltpu.SemaphoreType.DMA,
    ],
    compiler_params=pltpu.CompilerParams(use_tc_tiling_on_sc=False),
)
def _soft_prompt_embed(ids_hbm, table_hbm, prompt_hbm, out_hbm,
                       idx_v, rows_v, prm_v, gsem, psem):
    wid = lax.axis_index("s") * _NC + lax.axis_index("c")
    base = wid * _PER_W
    batch = base // _T
    out_base = base + (batch + 1) * _NP

    pltpu.sync_copy(ids_hbm.at[wid], idx_v)
    for j in range(_N_CHUNKS):
        pltpu.async_copy(
            table_hbm.at[idx_v.at[j]],
            rows_v.at[pl.ds(j * _IDX_CHUNK, _IDX_CHUNK)],
            gsem,
        )

    @pl.when(wid < _B)
    def _():
        copy = pltpu.async_copy(prompt_hbm, prm_v, psem)
        copy.wait()
        pltpu.sync_copy(prm_v, out_hbm.at[pl.ds(wid * (_NP + _T), _NP)])

    for j in range(_N_CHUNKS):
        pltpu.make_async_copy(
            table_hbm.at[idx_v.at[j]],
            rows_v.at[pl.ds(j * _IDX_CHUNK, _IDX_CHUNK)],
            gsem,
        ).wait()
    pltpu.sync_copy(rows_v, out_hbm.at[pl.ds(out_base, _PER_W)])


def kernel(input_ids, table, soft_prompt):
    ids_flat = input_ids.astype(jnp.int32).reshape(_NW, _N_CHUNKS, _IDX_CHUNK)
    out = _soft_prompt_embed(ids_flat, table, soft_prompt)
    return out.reshape(_B, _NP + _T, _D)
